# baseline (speedup 1.0000x reference)
import functools

import jax
import jax.numpy as jnp
from jax import lax
from jax.experimental import pallas as pl
from jax.experimental.pallas import tpu as pltpu
from jax.experimental.pallas import tpu_sc as plsc

N_FACTORS = 32
BATCH = 16384
N_ROWS = 1000000
NC = 2
NS = 16
L = 16
NW = NC * NS
B_PER_W = BATCH // NW
IDX_CHUNK = 128
N_CHUNKS = B_PER_W // IDX_CHUNK


def _body(user_hbm, item_hbm, ufT_hbm, ifT_hbm, out_hbm,
          uidx_v, iidx_v, ubufT, ibufT, out_v, sem):
    wid = lax.axis_index("s") * NC + lax.axis_index("c")
    base = wid * B_PER_W

    for j in range(N_CHUNKS):
        hsl = pl.ds(base + j * IDX_CHUNK, IDX_CHUNK)
        pltpu.sync_copy(user_hbm.at[hsl], uidx_v.at[j])
        pltpu.sync_copy(item_hbm.at[hsl], iidx_v.at[j])

    for j in range(N_CHUNKS):
        for i in range(IDX_CHUNK // L):
            sl = (j, pl.ds(i * L, L))
            uidx_v[sl] = uidx_v[sl] - 1
            iidx_v[sl] = iidx_v[sl] - 1

    for j in range(N_CHUNKS):
        copies = []
        for f in range(N_FACTORS):
            copies.append(pltpu.async_copy(
                ufT_hbm.at[f].at[uidx_v.at[j]], ubufT.at[f], sem))
            copies.append(pltpu.async_copy(
                ifT_hbm.at[f].at[iidx_v.at[j]], ibufT.at[f], sem))
        for c in copies:
            c.wait()

        def group(g, carry, j=j):
            gsl = pl.ds(g * L, L)
            acc = jnp.zeros((L,), jnp.float32)
            for f in range(N_FACTORS):
                acc = acc + ubufT[(f, gsl)] * ibufT[(f, gsl)]
            out_v[pl.ds(j * IDX_CHUNK + g * L, L)] = acc * 5.0
            return carry

        lax.fori_loop(0, IDX_CHUNK // L, group, 0)

    pltpu.sync_copy(out_v.at[...], out_hbm.at[pl.ds(base, B_PER_W)])


@jax.jit
def _mf_forward(user, item, ufT, ifT):
    mesh = plsc.VectorSubcoreMesh(core_axis_name="c", subcore_axis_name="s")
    f = pl.kernel(
        _body,
        mesh=mesh,
        out_type=jax.ShapeDtypeStruct((BATCH,), jnp.float32),
        scratch_types=[
            pltpu.VMEM((N_CHUNKS, IDX_CHUNK), jnp.int32),
            pltpu.VMEM((N_CHUNKS, IDX_CHUNK), jnp.int32),
            pltpu.VMEM((N_FACTORS, IDX_CHUNK), jnp.float32),
            pltpu.VMEM((N_FACTORS, IDX_CHUNK), jnp.float32),
            pltpu.VMEM((B_PER_W,), jnp.float32),
            pltpu.SemaphoreType.DMA,
        ],
        compiler_params=pltpu.CompilerParams(
            needs_layout_passes=False, use_tc_tiling_on_sc=False),
    )
    return f(user, item, ufT, ifT)


def kernel(user, item, user_factors, item_factors):
    return _mf_forward(user, item, user_factors.T, item_factors.T)


# speedup vs baseline: 5.5349x; 5.5349x over previous
import functools

import jax
import jax.numpy as jnp
from jax import lax
from jax.experimental import pallas as pl
from jax.experimental.pallas import tpu as pltpu
from jax.experimental.pallas import tpu_sc as plsc

N_FACTORS = 32
BATCH = 16384
N_ROWS = 1000000
ROW_W = 128
NC = 2
NS = 16
L = 16
NW = NC * NS
B_PER_W = BATCH // NW
IDX_CHUNK = 128
N_CHUNKS = B_PER_W // IDX_CHUNK


def _body(user_hbm, item_hbm, uf_hbm, if_hbm, out_hbm,
          uidx_v, iidx_v, ubuf, ibuf, out_v, sem):
    wid = lax.axis_index("s") * NC + lax.axis_index("c")
    base = wid * B_PER_W

    for j in range(N_CHUNKS):
        hsl = pl.ds(base + j * IDX_CHUNK, IDX_CHUNK)
        pltpu.sync_copy(user_hbm.at[hsl], uidx_v.at[j])
        pltpu.sync_copy(item_hbm.at[hsl], iidx_v.at[j])

    for j in range(N_CHUNKS):
        for i in range(IDX_CHUNK // L):
            sl = (j, pl.ds(i * L, L))
            uidx_v[sl] = uidx_v[sl] - 1
            iidx_v[sl] = iidx_v[sl] - 1

    lanes = lax.iota(jnp.int32, L)

    for j in range(N_CHUNKS):
        cu = pltpu.async_copy(uf_hbm.at[uidx_v.at[j]], ubuf.at[...], sem)
        ci = pltpu.async_copy(if_hbm.at[iidx_v.at[j]], ibuf.at[...], sem)
        cu.wait()
        ci.wait()

        def group(g, carry, j=j):
            rows = g * L + lanes
            acc = jnp.zeros((L,), jnp.float32)
            for d in range(N_FACTORS):
                dcol = jnp.full((L,), d, jnp.int32)
                uv = plsc.load_gather(ubuf, [rows, dcol])
                iv = plsc.load_gather(ibuf, [rows, dcol])
                acc = acc + uv * iv
            out_v[pl.ds(j * IDX_CHUNK + g * L, L)] = acc * 5.0
            return carry

        lax.fori_loop(0, IDX_CHUNK // L, group, 0)

    pltpu.sync_copy(out_v.at[...], out_hbm.at[pl.ds(base, B_PER_W)])


@jax.jit
def _mf_forward(user, item, uf_padded, if_padded):
    mesh = plsc.VectorSubcoreMesh(core_axis_name="c", subcore_axis_name="s")
    f = pl.kernel(
        _body,
        mesh=mesh,
        out_type=jax.ShapeDtypeStruct((BATCH,), jnp.float32),
        scratch_types=[
            pltpu.VMEM((N_CHUNKS, IDX_CHUNK), jnp.int32),
            pltpu.VMEM((N_CHUNKS, IDX_CHUNK), jnp.int32),
            pltpu.VMEM((IDX_CHUNK, ROW_W), jnp.float32),
            pltpu.VMEM((IDX_CHUNK, ROW_W), jnp.float32),
            pltpu.VMEM((B_PER_W,), jnp.float32),
            pltpu.SemaphoreType.DMA,
        ],
        compiler_params=pltpu.CompilerParams(needs_layout_passes=False),
    )
    return f(user, item, uf_padded, if_padded)


def kernel(user, item, user_factors, item_factors):
    uf_padded = jnp.pad(user_factors, ((0, 0), (0, ROW_W - N_FACTORS)))
    if_padded = jnp.pad(item_factors, ((0, 0), (0, ROW_W - N_FACTORS)))
    return _mf_forward(user, item, uf_padded, if_padded)


# speedup vs baseline: 5.6338x; 1.0179x over previous
import functools

import jax
import jax.numpy as jnp
from jax import lax
from jax.experimental import pallas as pl
from jax.experimental.pallas import tpu as pltpu
from jax.experimental.pallas import tpu_sc as plsc

N_FACTORS = 32
BATCH = 16384
NC = 2
NS = 16
L = 16
NW = NC * NS
B_PER_W = BATCH // NW
IDX_CHUNK = 128
N_CHUNKS = B_PER_W // IDX_CHUNK


def _body(user_hbm, item_hbm, uf_hbm, if_hbm, out_hbm,
          uidx_v, iidx_v, urows_v, irows_v, out_v, sem):
    wid = lax.axis_index("s") * NC + lax.axis_index("c")
    base = wid * B_PER_W

    for j in range(N_CHUNKS):
        hsl = pl.ds(base + j * IDX_CHUNK, IDX_CHUNK)
        pltpu.sync_copy(user_hbm.at[hsl], uidx_v.at[j])
        pltpu.sync_copy(item_hbm.at[hsl], iidx_v.at[j])

    for j in range(N_CHUNKS):
        for i in range(IDX_CHUNK // L):
            sl = (j, pl.ds(i * L, L))
            uidx_v[sl] = uidx_v[sl] - 1
            iidx_v[sl] = iidx_v[sl] - 1

    copies = []
    for j in range(N_CHUNKS):
        rsl = pl.ds(j * IDX_CHUNK, IDX_CHUNK)
        copies.append(pltpu.async_copy(uf_hbm.at[uidx_v.at[j]],
                                       urows_v.at[rsl], sem))
        copies.append(pltpu.async_copy(if_hbm.at[iidx_v.at[j]],
                                       irows_v.at[rsl], sem))
    for c in copies:
        c.wait()

    lanes = lax.iota(jnp.int32, L)

    def group(g, carry):
        rows = g * L + lanes
        acc = jnp.zeros((L,), jnp.float32)
        for d in range(N_FACTORS):
            dcol = jnp.full((L,), d, jnp.int32)
            uv = plsc.load_gather(urows_v, [rows, dcol])
            iv = plsc.load_gather(irows_v, [rows, dcol])
            acc = acc + uv * iv
        out_v[pl.ds(g * L, L)] = acc * 5.0
        return carry

    lax.fori_loop(0, B_PER_W // L, group, 0)

    pltpu.sync_copy(out_v.at[...], out_hbm.at[pl.ds(base, B_PER_W)])


@jax.jit
def _mf_forward(user, item, user_factors, item_factors):
    mesh = plsc.VectorSubcoreMesh(core_axis_name="c", subcore_axis_name="s")
    f = pl.kernel(
        _body,
        mesh=mesh,
        out_type=jax.ShapeDtypeStruct((BATCH,), jnp.float32),
        scratch_types=[
            pltpu.VMEM((N_CHUNKS, IDX_CHUNK), jnp.int32),
            pltpu.VMEM((N_CHUNKS, IDX_CHUNK), jnp.int32),
            pltpu.VMEM((B_PER_W, N_FACTORS), jnp.float32),
            pltpu.VMEM((B_PER_W, N_FACTORS), jnp.float32),
            pltpu.VMEM((B_PER_W,), jnp.float32),
            pltpu.SemaphoreType.DMA,
        ],
        compiler_params=pltpu.CompilerParams(
            needs_layout_passes=False, use_tc_tiling_on_sc=False),
    )
    return f(user, item, user_factors, item_factors)


def kernel(user, item, user_factors, item_factors):
    return _mf_forward(user, item, user_factors, item_factors)
